# baseline (speedup 1.0000x reference)
import functools

import jax
import jax.numpy as jnp
from jax import lax
from jax.experimental import pallas as pl
from jax.experimental.pallas import tpu as pltpu

try:
    from jax.experimental.pallas import tpu_sc as plsc
    _HAS_SC = True
except ImportError:
    _HAS_SC = False

N = 10000
NP = 10240
E = 160000
P = 100000
H = 512
HEADS = 8
DH = H // HEADS
NC = 2
NS = 16
NW = NC * NS
PP = 102400
EPAD = 163840
ECH = 160
PCH = 64


def _gelu(x):
    return 0.5 * x * (1.0 + lax.erf(x * 0.7071067811865476))


def _mm_body(x_ref, w_ref, b_ref, o_ref):
    o_ref[...] = (
        jnp.dot(x_ref[...], w_ref[...], preferred_element_type=jnp.float32)
        + b_ref[...]
    )


def tc_matmul(x, W, b, bm=512):
    M, K = x.shape
    _, No = W.shape
    return pl.pallas_call(
        _mm_body,
        grid=(M // bm,),
        in_specs=[
            pl.BlockSpec((bm, K), lambda i: (i, 0)),
            pl.BlockSpec((K, No), lambda i: (0, 0)),
            pl.BlockSpec((1, No), lambda i: (0, 0)),
        ],
        out_specs=pl.BlockSpec((bm, No), lambda i: (i, 0)),
        out_shape=jax.ShapeDtypeStruct((M, No), jnp.float32),
    )(x, W, b.reshape(1, -1))


def _mm2_body(x_ref, wa_ref, wb_ref, oa_ref, ob_ref):
    x = x_ref[...]
    oa_ref[...] = jnp.dot(x, wa_ref[...], preferred_element_type=jnp.float32)
    ob_ref[...] = jnp.dot(x, wb_ref[...], preferred_element_type=jnp.float32)


def tc_matmul2(x, Wa, Wb, bm=512):
    M, K = x.shape
    _, No = Wa.shape
    return pl.pallas_call(
        _mm2_body,
        grid=(M // bm,),
        in_specs=[
            pl.BlockSpec((bm, K), lambda i: (i, 0)),
            pl.BlockSpec((K, No), lambda i: (0, 0)),
            pl.BlockSpec((K, No), lambda i: (0, 0)),
        ],
        out_specs=[
            pl.BlockSpec((bm, No), lambda i: (i, 0)),
            pl.BlockSpec((bm, No), lambda i: (i, 0)),
        ],
        out_shape=[
            jax.ShapeDtypeStruct((M, No), jnp.float32),
            jax.ShapeDtypeStruct((M, No), jnp.float32),
        ],
    )(x, Wa, Wb)


def _group_mats():
    col = lax.broadcasted_iota(jnp.int32, (H, HEADS), 0) // DH
    hh = lax.broadcasted_iota(jnp.int32, (H, HEADS), 1)
    G = (col == hh).astype(jnp.float32)
    return G, G.T


def _edge_body(qd_ref, ks_ref, vs_ref, e_ref, m0_ref, m1_ref, m2_ref, m3_ref):
    G, Gt = _group_mats()
    p = qd_ref[...] * ks_ref[...]
    d = jnp.dot(p, G, preferred_element_type=jnp.float32) * (1.0 / 8.0)
    e = jnp.exp(d)
    e_ref[...] = jnp.concatenate(
        [e, jnp.zeros((e.shape[0], 120), jnp.float32)], axis=1)
    scale = jnp.dot(e, Gt, preferred_element_type=jnp.float32)
    msg = vs_ref[...] * scale
    m0_ref[...] = msg[:, 0:128]
    m1_ref[...] = msg[:, 128:256]
    m2_ref[...] = msg[:, 256:384]
    m3_ref[...] = msg[:, 384:512]


def tc_edge_math(qd, ks, vs, eb=1024):
    return pl.pallas_call(
        _edge_body,
        grid=(EPAD // eb,),
        in_specs=[pl.BlockSpec((eb, H), lambda i: (i, 0))] * 3,
        out_specs=[pl.BlockSpec((eb, 128), lambda i: (i, 0))] * 5,
        out_shape=[jax.ShapeDtypeStruct((EPAD, 128), jnp.float32)] * 5,
    )(qd, ks, vs)


def _ln_body(num_ref, den_ref, skip_ref, xres_ref, g_ref, b_ref, o_ref):
    _, Gt = _group_mats()
    num = num_ref[0] + num_ref[1]
    den = den_ref[0] + den_ref[1]
    recip = 1.0 / (den[:, :HEADS] + 1e-16)
    scale = jnp.dot(recip, Gt, preferred_element_type=jnp.float32)
    out = num * scale + skip_ref[...]
    mu = jnp.mean(out, axis=-1, keepdims=True)
    var = jnp.mean((out - mu) ** 2, axis=-1, keepdims=True)
    out = (out - mu) * lax.rsqrt(var + 1e-5) * g_ref[...] + b_ref[...]
    o_ref[...] = _gelu(out + xres_ref[...])


def tc_ln_gelu(numer2, denom2, skip, xres, ln_g, ln_b, bn=1024):
    return pl.pallas_call(
        _ln_body,
        grid=(NP // bn,),
        in_specs=[
            pl.BlockSpec((2, bn, H), lambda i: (0, i, 0)),
            pl.BlockSpec((2, bn, 128), lambda i: (0, i, 0)),
            pl.BlockSpec((bn, H), lambda i: (i, 0)),
            pl.BlockSpec((bn, H), lambda i: (i, 0)),
            pl.BlockSpec((1, H), lambda i: (0, 0)),
            pl.BlockSpec((1, H), lambda i: (0, 0)),
        ],
        out_specs=pl.BlockSpec((bn, H), lambda i: (i, 0)),
        out_shape=jax.ShapeDtypeStruct((NP, H), jnp.float32),
    )(numer2, denom2, skip, xres, ln_g.reshape(1, -1), ln_b.reshape(1, -1))


def _pair_body(ga_ref, gb_ref, pf_ref, w1c_ref, b1_ref, w2_ref, b2_ref,
               w3_ref, b3_ref, o_ref):
    h1 = (
        ga_ref[...] + gb_ref[...]
        + jnp.dot(pf_ref[...], w1c_ref[...], preferred_element_type=jnp.float32)
        + b1_ref[...]
    )
    h1 = _gelu(h1)
    h2 = jnp.dot(h1, w2_ref[...], preferred_element_type=jnp.float32) + b2_ref[...]
    h2 = _gelu(h2)
    o_ref[...] = jnp.sum(h2 * w3_ref[...], axis=-1, keepdims=True) + b3_ref[...]


def tc_pair_mlp(ga, gb, pf, W1c, b1, W2, b2, W3, b3, bp=512):
    H2 = 2 * H
    return pl.pallas_call(
        _pair_body,
        grid=(PP // bp,),
        in_specs=[
            pl.BlockSpec((bp, H2), lambda i: (i, 0)),
            pl.BlockSpec((bp, H2), lambda i: (i, 0)),
            pl.BlockSpec((bp, 16), lambda i: (i, 0)),
            pl.BlockSpec((16, H2), lambda i: (0, 0)),
            pl.BlockSpec((1, H2), lambda i: (0, 0)),
            pl.BlockSpec((H2, H), lambda i: (0, 0)),
            pl.BlockSpec((1, H), lambda i: (0, 0)),
            pl.BlockSpec((1, H), lambda i: (0, 0)),
            pl.BlockSpec((1, 1), lambda i: (0, 0)),
        ],
        out_specs=pl.BlockSpec((bp, 1), lambda i: (i, 0)),
        out_shape=jax.ShapeDtypeStruct((PP, 1), jnp.float32),
    )(ga, gb, pf, W1c, b1.reshape(1, -1), W2, b2.reshape(1, -1),
      W3.reshape(1, -1), b3.reshape(1, 1))


def sc_gather(table, idx, chunk):
    V, D = table.shape
    (B,) = idx.shape
    b_per_w = B // NW
    n_iter = b_per_w // chunk
    mesh = plsc.VectorSubcoreMesh(core_axis_name="c", subcore_axis_name="s")

    @functools.partial(
        pl.kernel,
        mesh=mesh,
        out_type=jax.ShapeDtypeStruct((B, D), jnp.float32),
        scratch_types=[
            pltpu.VMEM((chunk,), jnp.int32),
            pltpu.VMEM((chunk, D), jnp.float32),
            pltpu.SemaphoreType.DMA,
        ],
    )
    def k(table_hbm, idx_hbm, out_hbm, idx_v, rows_v, sem):
        wid = lax.axis_index("s") * NC + lax.axis_index("c")
        base = wid * b_per_w

        def body(i, _):
            off = base + i * chunk
            pltpu.sync_copy(idx_hbm.at[pl.ds(off, chunk)], idx_v)
            pltpu.async_copy(table_hbm.at[idx_v], rows_v, sem).wait()
            pltpu.sync_copy(rows_v, out_hbm.at[pl.ds(off, chunk)])
            return 0

        lax.fori_loop(0, n_iter, body, 0)

    return k(table, idx)


def sc_scatter_add(vals, idx, D, chunk):
    (B, Dv) = vals.shape
    b_per_w = B // NW
    n_iter = b_per_w // chunk
    rows_per_tile = NP // NS
    zrows = 32
    NB = 2
    mesh = plsc.VectorSubcoreMesh(core_axis_name="c", subcore_axis_name="s")

    @functools.partial(
        pl.kernel,
        mesh=mesh,
        out_type=jax.ShapeDtypeStruct((NC, NP, D), jnp.float32),
        scratch_types=(
            [pltpu.VMEM((chunk,), jnp.int32)] * NB
            + [
                pltpu.VMEM((NB, chunk, D), jnp.float32),
                pltpu.VMEM((zrows, D), jnp.float32),
                pltpu.VMEM_SHARED((NP, D), jnp.float32),
            ]
            + [pltpu.SemaphoreType.DMA] * (2 * NB)
        ),
    )
    def k(vals_hbm, idx_hbm, out_hbm, *scr):
        idx_vs = scr[:NB]
        vals_v, zero_v, shared = scr[NB:NB + 3]
        lsems = scr[NB + 3:NB + 3 + NB]
        asems = scr[NB + 3 + NB:]
        c = lax.axis_index("c")
        s = lax.axis_index("s")
        wid = s * NC + c
        base = wid * b_per_w

        def zbody(i, _):
            r = i // (D // 16)
            col = (i % (D // 16)) * 16
            zero_v[r, pl.ds(col, 16)] = jnp.zeros((16,), jnp.float32)
            return 0

        lax.fori_loop(0, zrows * (D // 16), zbody, 0)

        def zcopy(i, _):
            pltpu.sync_copy(
                zero_v,
                shared.at[pl.ds(s * rows_per_tile + i * zrows, zrows)],
            )
            return 0

        lax.fori_loop(0, rows_per_tile // zrows, zcopy, 0)
        plsc.subcore_barrier()

        def group(g, _):
            for b in range(NB):
                cc = g * NB + b

                @pl.when(cc < n_iter)
                def _():
                    off = base + cc * chunk
                    pltpu.async_copy(
                        idx_hbm.at[pl.ds(off, chunk)], idx_vs[b], lsems[b]
                    )
                    pltpu.async_copy(
                        vals_hbm.at[pl.ds(off, chunk)], vals_v.at[b], lsems[b]
                    )

            for b in range(NB):
                cc = g * NB + b

                @pl.when(cc < n_iter)
                def _():
                    off = base + cc * chunk
                    pltpu.make_async_copy(
                        idx_hbm.at[pl.ds(off, chunk)], idx_vs[b], lsems[b]
                    ).wait()
                    pltpu.make_async_copy(
                        vals_hbm.at[pl.ds(off, chunk)], vals_v.at[b], lsems[b]
                    ).wait()
                    pltpu.async_copy(
                        vals_v.at[b], shared.at[idx_vs[b]], asems[b], add=True
                    )

            for b in range(NB):
                cc = g * NB + b

                @pl.when(cc < n_iter)
                def _():
                    pltpu.make_async_copy(
                        vals_v.at[b], shared.at[idx_vs[b]], asems[b]
                    ).wait()

            return 0

        lax.fori_loop(0, (n_iter + NB - 1) // NB, group, 0)
        plsc.subcore_barrier()
        pltpu.sync_copy(
            shared.at[pl.ds(s * rows_per_tile, rows_per_tile)],
            out_hbm.at[c, pl.ds(s * rows_per_tile, rows_per_tile)],
        )

    return k(vals, idx)


def kernel(atom_features, edge_index, pair_indices, pair_features, params):
    src = edge_index[0]
    dst = edge_index[1]
    src_p = jnp.concatenate([src, jnp.zeros((EPAD - E,), jnp.int32)])
    dst_p = jnp.concatenate(
        [dst, N + (jnp.arange(EPAD - E, dtype=jnp.int32) % (NP - N))])


    af = jnp.pad(atom_features, ((0, NP - N), (0, 0)))
    x = tc_matmul(af, params['emb_W'], params['emb_b'])

    for lp in params['layers']:
        Wcat = jnp.concatenate([lp['Wq'], lp['Wk'], lp['Wv'], lp['Wskip']], axis=1)
        bcat = jnp.concatenate([lp['bq'], lp['bk'], lp['bv'], lp['bskip']])
        proj = tc_matmul(x, Wcat, bcat)
        q = proj[:, 0:H]
        kk = proj[:, H:2 * H]
        v = proj[:, 2 * H:3 * H]
        skip = proj[:, 3 * H:4 * H]

        qd = sc_gather(q, dst_p, chunk=ECH)
        ks = sc_gather(kk, src_p, chunk=ECH)
        vs = sc_gather(v, src_p, chunk=ECH)

        e_pad, m0, m1, m2, m3 = tc_edge_math(qd, ks, vs)

        denom2 = sc_scatter_add(e_pad, dst_p, D=128, chunk=ECH)
        n0 = sc_scatter_add(m0, dst_p, D=128, chunk=ECH)
        n1 = sc_scatter_add(m1, dst_p, D=128, chunk=ECH)
        n2 = sc_scatter_add(m2, dst_p, D=128, chunk=ECH)
        n3 = sc_scatter_add(m3, dst_p, D=128, chunk=ECH)
        numer2 = jnp.concatenate([n0, n1, n2, n3], axis=2)

        x = tc_ln_gelu(numer2, denom2, skip, x, lp['ln_g'], lp['ln_b'])

    (W1, b1), (W2, b2), (W3, b3) = params['mlp']
    W1a = W1[0:H]
    W1b = W1[H:2 * H]
    W1c = W1[2 * H:]
    xa, xb = tc_matmul2(x, W1a, W1b)

    i0 = jnp.pad(pair_indices[:, 0], (0, PP - P))
    i1 = jnp.pad(pair_indices[:, 1], (0, PP - P))
    pf = jnp.pad(pair_features, ((0, PP - P), (0, 0)))

    ga = sc_gather(xa, i0, chunk=PCH)
    gb = sc_gather(xb, i1, chunk=PCH)

    out = tc_pair_mlp(ga, gb, pf, W1c, b1, W2, b2, W3, b3)
    return out[:P]


# speedup vs baseline: 1.2223x; 1.2223x over previous
import functools

import jax
import jax.numpy as jnp
from jax import lax
from jax.experimental import pallas as pl
from jax.experimental.pallas import tpu as pltpu

try:
    from jax.experimental.pallas import tpu_sc as plsc
    _HAS_SC = True
except ImportError:
    _HAS_SC = False

N = 10000
NP = 10240
E = 160000
P = 100000
H = 512
HEADS = 8
DH = H // HEADS
NC = 2
NS = 16
NW = NC * NS
PP = 102400
ECH = 200
PCH = 64


def _gelu(x):
    return 0.5 * x * (1.0 + lax.erf(x * 0.7071067811865476))


def _mm_body(x_ref, w_ref, b_ref, o_ref):
    o_ref[...] = (
        jnp.dot(x_ref[...], w_ref[...], preferred_element_type=jnp.float32)
        + b_ref[...]
    )


def tc_matmul(x, W, b, bm=512):
    M, K = x.shape
    _, No = W.shape
    return pl.pallas_call(
        _mm_body,
        grid=(M // bm,),
        in_specs=[
            pl.BlockSpec((bm, K), lambda i: (i, 0)),
            pl.BlockSpec((K, No), lambda i: (0, 0)),
            pl.BlockSpec((1, No), lambda i: (0, 0)),
        ],
        out_specs=pl.BlockSpec((bm, No), lambda i: (i, 0)),
        out_shape=jax.ShapeDtypeStruct((M, No), jnp.float32),
    )(x, W, b.reshape(1, -1))


def _mm2_body(x_ref, wa_ref, wb_ref, oa_ref, ob_ref):
    x = x_ref[...]
    oa_ref[...] = jnp.dot(x, wa_ref[...], preferred_element_type=jnp.float32)
    ob_ref[...] = jnp.dot(x, wb_ref[...], preferred_element_type=jnp.float32)


def tc_matmul2(x, Wa, Wb, bm=512):
    M, K = x.shape
    _, No = Wa.shape
    return pl.pallas_call(
        _mm2_body,
        grid=(M // bm,),
        in_specs=[
            pl.BlockSpec((bm, K), lambda i: (i, 0)),
            pl.BlockSpec((K, No), lambda i: (0, 0)),
            pl.BlockSpec((K, No), lambda i: (0, 0)),
        ],
        out_specs=[
            pl.BlockSpec((bm, No), lambda i: (i, 0)),
            pl.BlockSpec((bm, No), lambda i: (i, 0)),
        ],
        out_shape=[
            jax.ShapeDtypeStruct((M, No), jnp.float32),
            jax.ShapeDtypeStruct((M, No), jnp.float32),
        ],
    )(x, Wa, Wb)


def _group_mats():
    col = lax.broadcasted_iota(jnp.int32, (H, HEADS), 0) // DH
    hh = lax.broadcasted_iota(jnp.int32, (H, HEADS), 1)
    G = (col == hh).astype(jnp.float32)
    return G, G.T


def _edge_body(qd_ref, ks_ref, vs_ref, e_ref, m0_ref, m1_ref, m2_ref, m3_ref):
    G, Gt = _group_mats()
    p = qd_ref[...] * ks_ref[...]
    d = jnp.dot(p, G, preferred_element_type=jnp.float32) * (1.0 / 8.0)
    e = jnp.exp(d)
    e_ref[...] = jnp.concatenate(
        [e, jnp.zeros((e.shape[0], 120), jnp.float32)], axis=1)
    scale = jnp.dot(e, Gt, preferred_element_type=jnp.float32)
    msg = vs_ref[...] * scale
    m0_ref[...] = msg[:, 0:128]
    m1_ref[...] = msg[:, 128:256]
    m2_ref[...] = msg[:, 256:384]
    m3_ref[...] = msg[:, 384:512]


def tc_edge_math(qd, ks, vs, eb=1000):
    return pl.pallas_call(
        _edge_body,
        grid=(E // eb,),
        in_specs=[pl.BlockSpec((eb, H), lambda i: (i, 0))] * 3,
        out_specs=[pl.BlockSpec((eb, 128), lambda i: (i, 0))] * 5,
        out_shape=[jax.ShapeDtypeStruct((E, 128), jnp.float32)] * 5,
    )(qd, ks, vs)


def _ln_body(num_ref, den_ref, skip_ref, xres_ref, g_ref, b_ref, o_ref):
    _, Gt = _group_mats()
    num = num_ref[0] + num_ref[1]
    den = den_ref[0] + den_ref[1]
    recip = 1.0 / (den[:, :HEADS] + 1e-16)
    scale = jnp.dot(recip, Gt, preferred_element_type=jnp.float32)
    out = num * scale + skip_ref[...]
    mu = jnp.mean(out, axis=-1, keepdims=True)
    var = jnp.mean((out - mu) ** 2, axis=-1, keepdims=True)
    out = (out - mu) * lax.rsqrt(var + 1e-5) * g_ref[...] + b_ref[...]
    o_ref[...] = _gelu(out + xres_ref[...])


def tc_ln_gelu(numer2, denom2, skip, xres, ln_g, ln_b, bn=1024):
    return pl.pallas_call(
        _ln_body,
        grid=(NP // bn,),
        in_specs=[
            pl.BlockSpec((2, bn, H), lambda i: (0, i, 0)),
            pl.BlockSpec((2, bn, 128), lambda i: (0, i, 0)),
            pl.BlockSpec((bn, H), lambda i: (i, 0)),
            pl.BlockSpec((bn, H), lambda i: (i, 0)),
            pl.BlockSpec((1, H), lambda i: (0, 0)),
            pl.BlockSpec((1, H), lambda i: (0, 0)),
        ],
        out_specs=pl.BlockSpec((bn, H), lambda i: (i, 0)),
        out_shape=jax.ShapeDtypeStruct((NP, H), jnp.float32),
    )(numer2, denom2, skip, xres, ln_g.reshape(1, -1), ln_b.reshape(1, -1))


def _pair_body(ga_ref, gb_ref, pf_ref, w1c_ref, b1_ref, w2_ref, b2_ref,
               w3_ref, b3_ref, o_ref):
    h1 = (
        ga_ref[...] + gb_ref[...]
        + jnp.dot(pf_ref[...], w1c_ref[...], preferred_element_type=jnp.float32)
        + b1_ref[...]
    )
    h1 = _gelu(h1)
    h2 = jnp.dot(h1, w2_ref[...], preferred_element_type=jnp.float32) + b2_ref[...]
    h2 = _gelu(h2)
    o_ref[...] = jnp.sum(h2 * w3_ref[...], axis=-1, keepdims=True) + b3_ref[...]


def tc_pair_mlp(ga, gb, pf, W1c, b1, W2, b2, W3, b3, bp=512):
    H2 = 2 * H
    return pl.pallas_call(
        _pair_body,
        grid=(PP // bp,),
        in_specs=[
            pl.BlockSpec((bp, H2), lambda i: (i, 0)),
            pl.BlockSpec((bp, H2), lambda i: (i, 0)),
            pl.BlockSpec((bp, 16), lambda i: (i, 0)),
            pl.BlockSpec((16, H2), lambda i: (0, 0)),
            pl.BlockSpec((1, H2), lambda i: (0, 0)),
            pl.BlockSpec((H2, H), lambda i: (0, 0)),
            pl.BlockSpec((1, H), lambda i: (0, 0)),
            pl.BlockSpec((1, H), lambda i: (0, 0)),
            pl.BlockSpec((1, 1), lambda i: (0, 0)),
        ],
        out_specs=pl.BlockSpec((bp, 1), lambda i: (i, 0)),
        out_shape=jax.ShapeDtypeStruct((PP, 1), jnp.float32),
    )(ga, gb, pf, W1c, b1.reshape(1, -1), W2, b2.reshape(1, -1),
      W3.reshape(1, -1), b3.reshape(1, 1))


def sc_gather(table, idx, chunk):
    V, D = table.shape
    (B,) = idx.shape
    b_per_w = B // NW
    n_iter = b_per_w // chunk
    mesh = plsc.VectorSubcoreMesh(core_axis_name="c", subcore_axis_name="s")

    @functools.partial(
        pl.kernel,
        mesh=mesh,
        out_type=jax.ShapeDtypeStruct((B, D), jnp.float32),
        scratch_types=[
            pltpu.VMEM((chunk,), jnp.int32),
            pltpu.VMEM((chunk, D), jnp.float32),
            pltpu.SemaphoreType.DMA,
        ],
    )
    def k(table_hbm, idx_hbm, out_hbm, idx_v, rows_v, sem):
        wid = lax.axis_index("s") * NC + lax.axis_index("c")
        base = wid * b_per_w

        def body(i, _):
            off = base + i * chunk
            pltpu.sync_copy(idx_hbm.at[pl.ds(off, chunk)], idx_v)
            pltpu.async_copy(table_hbm.at[idx_v], rows_v, sem).wait()
            pltpu.sync_copy(rows_v, out_hbm.at[pl.ds(off, chunk)])
            return 0

        lax.fori_loop(0, n_iter, body, 0)

    return k(table, idx)


def sc_scatter_add(vals, idx, D, chunk):
    (B, Dv) = vals.shape
    b_per_w = B // NW
    n_iter = b_per_w // chunk
    rows_per_tile = NP // NS
    zrows = 32
    mesh = plsc.VectorSubcoreMesh(core_axis_name="c", subcore_axis_name="s")

    @functools.partial(
        pl.kernel,
        mesh=mesh,
        out_type=jax.ShapeDtypeStruct((NC, NP, D), jnp.float32),
        scratch_types=[
            pltpu.VMEM((chunk,), jnp.int32),
            pltpu.VMEM((chunk, D), jnp.float32),
            pltpu.VMEM((zrows, D), jnp.float32),
            pltpu.VMEM_SHARED((NP, D), jnp.float32),
            pltpu.SemaphoreType.DMA,
        ],
    )
    def k(vals_hbm, idx_hbm, out_hbm, idx_v, vals_v, zero_v, shared, sem):
        c = lax.axis_index("c")
        s = lax.axis_index("s")
        wid = s * NC + c
        base = wid * b_per_w

        def zbody(i, _):
            r = i // (D // 16)
            col = (i % (D // 16)) * 16
            zero_v[r, pl.ds(col, 16)] = jnp.zeros((16,), jnp.float32)
            return 0

        lax.fori_loop(0, zrows * (D // 16), zbody, 0)

        def zcopy(i, _):
            pltpu.sync_copy(
                zero_v,
                shared.at[pl.ds(s * rows_per_tile + i * zrows, zrows)],
            )
            return 0

        lax.fori_loop(0, rows_per_tile // zrows, zcopy, 0)
        plsc.subcore_barrier()

        def body(i, _):
            off = base + i * chunk
            pltpu.sync_copy(idx_hbm.at[pl.ds(off, chunk)], idx_v)
            pltpu.sync_copy(vals_hbm.at[pl.ds(off, chunk)], vals_v)
            pltpu.sync_copy(vals_v, shared.at[idx_v], add=True)
            return 0

        lax.fori_loop(0, n_iter, body, 0)
        plsc.subcore_barrier()
        pltpu.sync_copy(
            shared.at[pl.ds(s * rows_per_tile, rows_per_tile)],
            out_hbm.at[c, pl.ds(s * rows_per_tile, rows_per_tile)],
        )

    return k(vals, idx)


def kernel(atom_features, edge_index, pair_indices, pair_features, params):
    src = edge_index[0]
    dst = edge_index[1]


    af = jnp.pad(atom_features, ((0, NP - N), (0, 0)))
    x = tc_matmul(af, params['emb_W'], params['emb_b'])

    for lp in params['layers']:
        Wcat = jnp.concatenate([lp['Wq'], lp['Wk'], lp['Wv'], lp['Wskip']], axis=1)
        bcat = jnp.concatenate([lp['bq'], lp['bk'], lp['bv'], lp['bskip']])
        proj = tc_matmul(x, Wcat, bcat)
        q = proj[:, 0:H]
        kk = proj[:, H:2 * H]
        v = proj[:, 2 * H:3 * H]
        skip = proj[:, 3 * H:4 * H]

        qd = sc_gather(q, dst, chunk=ECH)
        ks = sc_gather(kk, src, chunk=ECH)
        vs = sc_gather(v, src, chunk=ECH)

        e_pad, m0, m1, m2, m3 = tc_edge_math(qd, ks, vs)

        denom2 = sc_scatter_add(e_pad, dst, D=128, chunk=ECH)
        n0 = sc_scatter_add(m0, dst, D=128, chunk=ECH)
        n1 = sc_scatter_add(m1, dst, D=128, chunk=ECH)
        n2 = sc_scatter_add(m2, dst, D=128, chunk=ECH)
        n3 = sc_scatter_add(m3, dst, D=128, chunk=ECH)
        numer2 = jnp.concatenate([n0, n1, n2, n3], axis=2)

        x = tc_ln_gelu(numer2, denom2, skip, x, lp['ln_g'], lp['ln_b'])

    (W1, b1), (W2, b2), (W3, b3) = params['mlp']
    W1a = W1[0:H]
    W1b = W1[H:2 * H]
    W1c = W1[2 * H:]
    xa, xb = tc_matmul2(x, W1a, W1b)

    i0 = jnp.pad(pair_indices[:, 0], (0, PP - P))
    i1 = jnp.pad(pair_indices[:, 1], (0, PP - P))
    pf = jnp.pad(pair_features, ((0, PP - P), (0, 0)))

    ga = sc_gather(xa, i0, chunk=PCH)
    gb = sc_gather(xb, i1, chunk=PCH)

    out = tc_pair_mlp(ga, gb, pf, W1c, b1, W2, b2, W3, b3)
    return out[:P]


# speedup vs baseline: 1.2562x; 1.0277x over previous
import functools

import jax
import jax.numpy as jnp
from jax import lax
from jax.experimental import pallas as pl
from jax.experimental.pallas import tpu as pltpu

try:
    from jax.experimental.pallas import tpu_sc as plsc
    _HAS_SC = True
except ImportError:
    _HAS_SC = False

N = 10000
NP = 10240
E = 160000
P = 100000
H = 512
HEADS = 8
DH = H // HEADS
NC = 2
NS = 16
NW = NC * NS
PP = 102400
ECH = 200
PCH = 64


def _gelu(x):
    return 0.5 * x * (1.0 + lax.erf(x * 0.7071067811865476))


def _mm_body(x_ref, w_ref, b_ref, o_ref):
    o_ref[...] = (
        jnp.dot(x_ref[...], w_ref[...], preferred_element_type=jnp.float32)
        + b_ref[...]
    )


def tc_matmul(x, W, b, bm=512):
    M, K = x.shape
    _, No = W.shape
    return pl.pallas_call(
        _mm_body,
        grid=(M // bm,),
        in_specs=[
            pl.BlockSpec((bm, K), lambda i: (i, 0)),
            pl.BlockSpec((K, No), lambda i: (0, 0)),
            pl.BlockSpec((1, No), lambda i: (0, 0)),
        ],
        out_specs=pl.BlockSpec((bm, No), lambda i: (i, 0)),
        out_shape=jax.ShapeDtypeStruct((M, No), jnp.float32),
    )(x, W, b.reshape(1, -1))


def _mm2_body(x_ref, wa_ref, wb_ref, oa_ref, ob_ref):
    x = x_ref[...]
    oa_ref[...] = jnp.dot(x, wa_ref[...], preferred_element_type=jnp.float32)
    ob_ref[...] = jnp.dot(x, wb_ref[...], preferred_element_type=jnp.float32)


def tc_matmul2(x, Wa, Wb, bm=512):
    M, K = x.shape
    _, No = Wa.shape
    return pl.pallas_call(
        _mm2_body,
        grid=(M // bm,),
        in_specs=[
            pl.BlockSpec((bm, K), lambda i: (i, 0)),
            pl.BlockSpec((K, No), lambda i: (0, 0)),
            pl.BlockSpec((K, No), lambda i: (0, 0)),
        ],
        out_specs=[
            pl.BlockSpec((bm, No), lambda i: (i, 0)),
            pl.BlockSpec((bm, No), lambda i: (i, 0)),
        ],
        out_shape=[
            jax.ShapeDtypeStruct((M, No), jnp.float32),
            jax.ShapeDtypeStruct((M, No), jnp.float32),
        ],
    )(x, Wa, Wb)


def _group_mats():
    col = lax.broadcasted_iota(jnp.int32, (H, HEADS), 0) // DH
    hh = lax.broadcasted_iota(jnp.int32, (H, HEADS), 1)
    G = (col == hh).astype(jnp.float32)
    return G, G.T


def _edge_body(qd_ref, ks_ref, vs_ref, e_ref, m0_ref, m1_ref, m2_ref, m3_ref):
    G, Gt = _group_mats()
    p = qd_ref[...] * ks_ref[...]
    d = jnp.dot(p, G, preferred_element_type=jnp.float32) * (1.0 / 8.0)
    e = jnp.exp(d)
    e_ref[...] = jnp.concatenate(
        [e, jnp.zeros((e.shape[0], 120), jnp.float32)], axis=1)
    scale = jnp.dot(e, Gt, preferred_element_type=jnp.float32)
    msg = vs_ref[...] * scale
    m0_ref[...] = msg[:, 0:128]
    m1_ref[...] = msg[:, 128:256]
    m2_ref[...] = msg[:, 256:384]
    m3_ref[...] = msg[:, 384:512]


def tc_edge_math(qd, ks, vs, eb=1000):
    return pl.pallas_call(
        _edge_body,
        grid=(E // eb,),
        in_specs=[pl.BlockSpec((eb, H), lambda i: (i, 0))] * 3,
        out_specs=[pl.BlockSpec((eb, 128), lambda i: (i, 0))] * 5,
        out_shape=[jax.ShapeDtypeStruct((E, 128), jnp.float32)] * 5,
    )(qd, ks, vs)


def _ln_body(num_ref, den_ref, skip_ref, xres_ref, g_ref, b_ref, o_ref):
    _, Gt = _group_mats()
    num = num_ref[0] + num_ref[1]
    den = den_ref[0] + den_ref[1]
    recip = 1.0 / (den[:, :HEADS] + 1e-16)
    scale = jnp.dot(recip, Gt, preferred_element_type=jnp.float32)
    out = num * scale + skip_ref[...]
    mu = jnp.mean(out, axis=-1, keepdims=True)
    var = jnp.mean((out - mu) ** 2, axis=-1, keepdims=True)
    out = (out - mu) * lax.rsqrt(var + 1e-5) * g_ref[...] + b_ref[...]
    o_ref[...] = _gelu(out + xres_ref[...])


def tc_ln_gelu(numer2, denom2, skip, xres, ln_g, ln_b, bn=1024):
    return pl.pallas_call(
        _ln_body,
        grid=(NP // bn,),
        in_specs=[
            pl.BlockSpec((2, bn, H), lambda i: (0, i, 0)),
            pl.BlockSpec((2, bn, 128), lambda i: (0, i, 0)),
            pl.BlockSpec((bn, H), lambda i: (i, 0)),
            pl.BlockSpec((bn, H), lambda i: (i, 0)),
            pl.BlockSpec((1, H), lambda i: (0, 0)),
            pl.BlockSpec((1, H), lambda i: (0, 0)),
        ],
        out_specs=pl.BlockSpec((bn, H), lambda i: (i, 0)),
        out_shape=jax.ShapeDtypeStruct((NP, H), jnp.float32),
    )(numer2, denom2, skip, xres, ln_g.reshape(1, -1), ln_b.reshape(1, -1))


def _pair_body(ga_ref, gb_ref, pf_ref, w1c_ref, b1_ref, w2_ref, b2_ref,
               w3_ref, b3_ref, o_ref):
    h1 = (
        ga_ref[...] + gb_ref[...]
        + jnp.dot(pf_ref[...], w1c_ref[...], preferred_element_type=jnp.float32)
        + b1_ref[...]
    )
    h1 = _gelu(h1)
    h2 = jnp.dot(h1, w2_ref[...], preferred_element_type=jnp.float32) + b2_ref[...]
    h2 = _gelu(h2)
    o_ref[...] = jnp.sum(h2 * w3_ref[...], axis=-1, keepdims=True) + b3_ref[...]


def tc_pair_mlp(ga, gb, pf, W1c, b1, W2, b2, W3, b3, bp=512, rows=PP):
    H2 = 2 * H
    return pl.pallas_call(
        _pair_body,
        grid=(rows // bp,),
        in_specs=[
            pl.BlockSpec((bp, H2), lambda i: (i, 0)),
            pl.BlockSpec((bp, H2), lambda i: (i, 0)),
            pl.BlockSpec((bp, 16), lambda i: (i, 0)),
            pl.BlockSpec((16, H2), lambda i: (0, 0)),
            pl.BlockSpec((1, H2), lambda i: (0, 0)),
            pl.BlockSpec((H2, H), lambda i: (0, 0)),
            pl.BlockSpec((1, H), lambda i: (0, 0)),
            pl.BlockSpec((1, H), lambda i: (0, 0)),
            pl.BlockSpec((1, 1), lambda i: (0, 0)),
        ],
        out_specs=pl.BlockSpec((bp, 1), lambda i: (i, 0)),
        out_shape=jax.ShapeDtypeStruct((rows, 1), jnp.float32),
    )(ga, gb, pf, W1c, b1.reshape(1, -1), W2, b2.reshape(1, -1),
      W3.reshape(1, -1), b3.reshape(1, 1))


def sc_gather(table, idx, chunk):
    V, D = table.shape
    (B,) = idx.shape
    b_per_w = B // NW
    n_iter = b_per_w // chunk
    mesh = plsc.VectorSubcoreMesh(core_axis_name="c", subcore_axis_name="s")

    @functools.partial(
        pl.kernel,
        mesh=mesh,
        out_type=jax.ShapeDtypeStruct((B, D), jnp.float32),
        scratch_types=[
            pltpu.VMEM((chunk,), jnp.int32),
            pltpu.VMEM((chunk, D), jnp.float32),
            pltpu.SemaphoreType.DMA,
        ],
    )
    def k(table_hbm, idx_hbm, out_hbm, idx_v, rows_v, sem):
        wid = lax.axis_index("s") * NC + lax.axis_index("c")
        base = wid * b_per_w

        def body(i, _):
            off = base + i * chunk
            pltpu.sync_copy(idx_hbm.at[pl.ds(off, chunk)], idx_v)
            pltpu.async_copy(table_hbm.at[idx_v], rows_v, sem).wait()
            pltpu.sync_copy(rows_v, out_hbm.at[pl.ds(off, chunk)])
            return 0

        lax.fori_loop(0, n_iter, body, 0)

    return k(table, idx)


def sc_scatter_add(vals, idx, D, chunk):
    (B, Dv) = vals.shape
    b_per_w = B // NW
    n_iter = b_per_w // chunk
    rows_per_tile = NP // NS
    zrows = 32
    mesh = plsc.VectorSubcoreMesh(core_axis_name="c", subcore_axis_name="s")

    @functools.partial(
        pl.kernel,
        mesh=mesh,
        out_type=jax.ShapeDtypeStruct((NC, NP, D), jnp.float32),
        scratch_types=[
            pltpu.VMEM((chunk,), jnp.int32),
            pltpu.VMEM((chunk, D), jnp.float32),
            pltpu.VMEM((zrows, D), jnp.float32),
            pltpu.VMEM_SHARED((NP, D), jnp.float32),
            pltpu.SemaphoreType.DMA,
        ],
    )
    def k(vals_hbm, idx_hbm, out_hbm, idx_v, vals_v, zero_v, shared, sem):
        c = lax.axis_index("c")
        s = lax.axis_index("s")
        wid = s * NC + c
        base = wid * b_per_w

        def zbody(i, _):
            r = i // (D // 16)
            col = (i % (D // 16)) * 16
            zero_v[r, pl.ds(col, 16)] = jnp.zeros((16,), jnp.float32)
            return 0

        lax.fori_loop(0, zrows * (D // 16), zbody, 0)

        def zcopy(i, _):
            pltpu.sync_copy(
                zero_v,
                shared.at[pl.ds(s * rows_per_tile + i * zrows, zrows)],
            )
            return 0

        lax.fori_loop(0, rows_per_tile // zrows, zcopy, 0)
        plsc.subcore_barrier()

        def body(i, _):
            off = base + i * chunk
            pltpu.sync_copy(idx_hbm.at[pl.ds(off, chunk)], idx_v)
            pltpu.sync_copy(vals_hbm.at[pl.ds(off, chunk)], vals_v)
            pltpu.sync_copy(vals_v, shared.at[idx_v], add=True)
            return 0

        lax.fori_loop(0, n_iter, body, 0)
        plsc.subcore_barrier()
        pltpu.sync_copy(
            shared.at[pl.ds(s * rows_per_tile, rows_per_tile)],
            out_hbm.at[c, pl.ds(s * rows_per_tile, rows_per_tile)],
        )

    return k(vals, idx)


def kernel(atom_features, edge_index, pair_indices, pair_features, params):
    src = edge_index[0]
    dst = edge_index[1]


    af = jnp.pad(atom_features, ((0, NP - N), (0, 0)))
    x = tc_matmul(af, params['emb_W'], params['emb_b'])

    for lp in params['layers']:
        Wcat = jnp.concatenate([lp['Wq'], lp['Wk'], lp['Wv'], lp['Wskip']], axis=1)
        bcat = jnp.concatenate([lp['bq'], lp['bk'], lp['bv'], lp['bskip']])
        proj = tc_matmul(x, Wcat, bcat)
        q = proj[:, 0:H]
        kk = proj[:, H:2 * H]
        v = proj[:, 2 * H:3 * H]
        skip = proj[:, 3 * H:4 * H]

        qd = sc_gather(q, dst, chunk=ECH)
        ks = sc_gather(kk, src, chunk=ECH)
        vs = sc_gather(v, src, chunk=ECH)

        e_pad, m0, m1, m2, m3 = tc_edge_math(qd, ks, vs)

        denom2 = sc_scatter_add(e_pad, dst, D=128, chunk=ECH)
        n0 = sc_scatter_add(m0, dst, D=128, chunk=ECH)
        n1 = sc_scatter_add(m1, dst, D=128, chunk=ECH)
        n2 = sc_scatter_add(m2, dst, D=128, chunk=ECH)
        n3 = sc_scatter_add(m3, dst, D=128, chunk=ECH)
        numer2 = jnp.concatenate([n0, n1, n2, n3], axis=2)

        x = tc_ln_gelu(numer2, denom2, skip, x, lp['ln_g'], lp['ln_b'])

    (W1, b1), (W2, b2), (W3, b3) = params['mlp']
    W1a = W1[0:H]
    W1b = W1[H:2 * H]
    W1c = W1[2 * H:]
    xa, xb = tc_matmul2(x, W1a, W1b)

    i0 = jnp.pad(pair_indices[:, 0], (0, PP - P))
    i1 = jnp.pad(pair_indices[:, 1], (0, PP - P))
    pf = jnp.pad(pair_features, ((0, PP - P), (0, 0)))

    HP = PP // 2
    outs = []
    for lo in (0, HP):
        ga = sc_gather(xa, i0[lo:lo + HP], chunk=PCH)
        gb = sc_gather(xb, i1[lo:lo + HP], chunk=PCH)
        outs.append(
            tc_pair_mlp(ga, gb, pf[lo:lo + HP], W1c, b1, W2, b2, W3, b3,
                        rows=HP))
    return jnp.concatenate(outs, axis=0)[:P]
